# baseline (speedup 1.0000x reference)
import functools

import jax
import jax.numpy as jnp
from jax import lax
from jax.experimental import pallas as pl
from jax.experimental.pallas import tpu as pltpu
from jax.experimental.pallas import tpu_sc as plsc

D = 768
L = 16
NC, NS = 2, 16
NW = NC * NS
CHUNK = 32


def _embed_body(word_hbm, posb_hbm, wi_hbm, pi_hbm, out_hbm,
                idxw_v, idxp_v, bufw0, bufw1, bufp0, bufp1,
                semw0, semw1, semp0, semp1, semst0, semst1):
    wid = lax.axis_index("s") * NC + lax.axis_index("c")
    rows_per_w = out_hbm.shape[0] // NW
    n_chunks = rows_per_w // CHUNK
    base = wid * rows_per_w

    bufw = (bufw0, bufw1)
    bufp = (bufp0, bufp1)
    semw = (semw0, semw1)
    semp = (semp0, semp1)
    semst = (semst0, semst1)

    pltpu.sync_copy(wi_hbm.at[pl.ds(base, rows_per_w)], idxw_v)
    pltpu.sync_copy(pi_hbm.at[pl.ds(base, rows_per_w)], idxp_v)

    def widx(k):
        return idxw_v.at[pl.ds(k * CHUNK, CHUNK)]

    def pidx(k):
        return idxp_v.at[pl.ds(k * CHUNK, CHUNK)]

    def fire_gathers(k, slot):
        pltpu.async_copy(word_hbm.at[widx(k)], bufw[slot], semw[slot])
        pltpu.async_copy(posb_hbm.at[pidx(k)], bufp[slot], semp[slot])

    fire_gathers(0, 0)

    for g in range(n_chunks):
        s = g % 2
        o = 1 - s
        pltpu.make_async_copy(word_hbm.at[widx(g)], bufw[s], semw[s]).wait()
        pltpu.make_async_copy(posb_hbm.at[pidx(g)], bufp[s], semp[s]).wait()
        if g >= 1:
            pltpu.make_async_copy(
                bufw[o], out_hbm.at[pl.ds(base + (g - 1) * CHUNK, CHUNK)],
                semst[o]).wait()
        if g + 1 < n_chunks:
            fire_gathers(g + 1, o)

        def add_row(r, _, s=s):
            for c in range(D // (2 * L)):
                v = bufp[s][r, pl.ds(c * L, L)]
                a = lax.bitcast_convert_type(v << 16, jnp.float32)
                b = lax.bitcast_convert_type(v & jnp.int32(-65536),
                                             jnp.float32)
                sl0 = pl.ds(c * 2 * L, L)
                sl1 = pl.ds(c * 2 * L + L, L)
                bufw[s][r, sl0] = bufw[s][r, sl0] + a
                bufw[s][r, sl1] = bufw[s][r, sl1] + b
            return _

        lax.fori_loop(0, CHUNK, add_row, 0)
        pltpu.async_copy(bufw[s], out_hbm.at[pl.ds(base + g * CHUNK, CHUNK)],
                         semst[s])

    s_last = (n_chunks - 1) % 2
    pltpu.make_async_copy(
        bufw[s_last],
        out_hbm.at[pl.ds(base + (n_chunks - 1) * CHUNK, CHUNK)],
        semst[s_last]).wait()


@functools.partial(jax.jit, static_argnums=())
def _embed(word_embeddings, pos_b, wi, pi):
    n = wi.shape[0]
    rows_per_w = n // NW
    mesh = plsc.VectorSubcoreMesh(core_axis_name="c", subcore_axis_name="s",
                                  num_cores=NC, num_subcores=NS)
    return pl.kernel(
        _embed_body,
        out_type=jax.ShapeDtypeStruct((n, D), jnp.float32),
        mesh=mesh,
        scratch_types=(
            [pltpu.VMEM((rows_per_w,), jnp.int32)] * 2
            + [pltpu.VMEM((CHUNK, D), jnp.float32)] * 2
            + [pltpu.VMEM((CHUNK, D // 2), jnp.int32)] * 2
            + [pltpu.SemaphoreType.DMA] * 6
        ),
    )(word_embeddings, pos_b, wi, pi)


def kernel(input_ids, position_ids, word_embeddings, position_embeddings):
    B, S = input_ids.shape
    P = position_embeddings.shape[0]
    wi = input_ids.reshape(-1).astype(jnp.int32)
    pi = position_ids.reshape(-1).astype(jnp.int32)
    t = position_embeddings.reshape(P, D // 32, 2, 16)
    lo = lax.bitcast_convert_type(t[:, :, 0, :].astype(jnp.bfloat16),
                                  jnp.uint16).astype(jnp.uint32)
    hi = lax.bitcast_convert_type(t[:, :, 1, :].astype(jnp.bfloat16),
                                  jnp.uint16).astype(jnp.uint32)
    pos_b = lax.bitcast_convert_type(lo | (hi << 16),
                                     jnp.int32).reshape(P, D // 2)
    out = _embed(word_embeddings, pos_b, wi, pi)
    return out.reshape(B, S, D)


# speedup vs baseline: 1.6415x; 1.6415x over previous
import functools

import jax
import jax.numpy as jnp
from jax import lax
from jax.experimental import pallas as pl
from jax.experimental.pallas import tpu as pltpu
from jax.experimental.pallas import tpu_sc as plsc

D = 768
L = 16
NC, NS = 2, 16
NW = NC * NS
CHUNK = 32


def _embed_body(word_hbm, pos_hbm, wi_hbm, pi_hbm, out_hbm,
                idxw_v, idxp_v, bufw0, bufw1, bufp0, bufp1,
                semw0, semw1, semp0, semp1, semst0, semst1):
    wid = lax.axis_index("s") * NC + lax.axis_index("c")
    rows_per_w = out_hbm.shape[0] // NW
    n_chunks = rows_per_w // CHUNK
    base = wid * rows_per_w

    bufw = (bufw0, bufw1)
    bufp = (bufp0, bufp1)
    semw = (semw0, semw1)
    semp = (semp0, semp1)
    semst = (semst0, semst1)

    pltpu.sync_copy(wi_hbm.at[pl.ds(base, rows_per_w)], idxw_v)
    pltpu.sync_copy(pi_hbm.at[pl.ds(base, rows_per_w)], idxp_v)

    def widx(k):
        return idxw_v.at[pl.ds(k * CHUNK, CHUNK)]

    def pidx(k):
        return idxp_v.at[pl.ds(k * CHUNK, CHUNK)]

    def fire_gathers(k, slot):
        pltpu.async_copy(word_hbm.at[widx(k)], bufw[slot], semw[slot])
        pltpu.async_copy(pos_hbm.at[pidx(k)], bufp[slot], semp[slot])

    fire_gathers(0, 0)

    for g in range(n_chunks):
        s = g % 2
        o = 1 - s
        pltpu.make_async_copy(word_hbm.at[widx(g)], bufw[s], semw[s]).wait()
        pltpu.make_async_copy(pos_hbm.at[pidx(g)], bufp[s], semp[s]).wait()
        if g >= 1:
            pltpu.make_async_copy(
                bufw[o], out_hbm.at[pl.ds(base + (g - 1) * CHUNK, CHUNK)],
                semst[o]).wait()
        if g + 1 < n_chunks:
            fire_gathers(g + 1, o)

        @functools.partial(plsc.parallel_loop, 0, CHUNK, unroll=2)
        def _(r, s=s):
            for c in range(D // L):
                sl = pl.ds(c * L, L)
                bufw[s][r, sl] = bufw[s][r, sl] + bufp[s][r, sl]

        pltpu.async_copy(bufw[s], out_hbm.at[pl.ds(base + g * CHUNK, CHUNK)],
                         semst[s])

    s_last = (n_chunks - 1) % 2
    pltpu.make_async_copy(
        bufw[s_last],
        out_hbm.at[pl.ds(base + (n_chunks - 1) * CHUNK, CHUNK)],
        semst[s_last]).wait()


@functools.partial(jax.jit, static_argnums=())
def _embed(word_embeddings, position_embeddings, wi, pi):
    n = wi.shape[0]
    rows_per_w = n // NW
    mesh = plsc.VectorSubcoreMesh(core_axis_name="c", subcore_axis_name="s",
                                  num_cores=NC, num_subcores=NS)
    return pl.kernel(
        _embed_body,
        out_type=jax.ShapeDtypeStruct((n, D), jnp.float32),
        mesh=mesh,
        scratch_types=(
            [pltpu.VMEM((rows_per_w,), jnp.int32)] * 2
            + [pltpu.VMEM((CHUNK, D), jnp.float32)] * 4
            + [pltpu.SemaphoreType.DMA] * 6
        ),
    )(word_embeddings, position_embeddings, wi, pi)


def kernel(input_ids, position_ids, word_embeddings, position_embeddings):
    B, S = input_ids.shape
    wi = input_ids.reshape(-1).astype(jnp.int32)
    pi = position_ids.reshape(-1).astype(jnp.int32)
    out = _embed(word_embeddings, position_embeddings, wi, pi)
    return out.reshape(B, S, D)
